# baseline (speedup 1.0000x reference)
import jax
import jax.numpy as jnp
from jax import lax
from jax.experimental import pallas as pl
from jax.experimental.pallas import tpu as pltpu
from jax.experimental.pallas import tpu_sc as plsc

F32 = jnp.float32

NB = 40000
NBP = 40960
D = 128
TE = 1280000
TEP = 1310720
PAD_DST = 40448

_SC_MESH = plsc.VectorSubcoreMesh(core_axis_name="c", subcore_axis_name="s")
_SC_PARAMS = pltpu.CompilerParams(use_tc_tiling_on_sc=False)


def _deg_body(dst_hbm, ones_hbm, zs_hbm, deg_hbm, didx, ones_v, vbuf, acc):
    c = lax.axis_index("c")
    s = lax.axis_index("s")
    base = s * 2560
    pltpu.sync_copy(ones_hbm, ones_v)
    pltpu.sync_copy(zs_hbm, vbuf)
    pltpu.sync_copy(vbuf, acc.at[pl.ds(base, 2560)])
    plsc.subcore_barrier()

    @pl.loop(0, 40)
    def _(i):
        e0 = c * (TEP // 2) + (s * 40 + i) * 1024
        pltpu.sync_copy(dst_hbm.at[pl.ds(e0, 1024)], didx)
        pltpu.sync_copy(ones_v, acc.at[didx], add=True)

    plsc.subcore_barrier()
    pltpu.sync_copy(acc.at[pl.ds(base, 2560)], vbuf)
    pltpu.sync_copy(vbuf, deg_hbm.at[c, pl.ds(base, 2560)])


_deg_call = pl.kernel(
    _deg_body,
    out_type=jax.ShapeDtypeStruct((2, NBP, 16), F32),
    mesh=_SC_MESH,
    scratch_types=[
        pltpu.VMEM((1024,), jnp.int32),
        pltpu.VMEM((1024, 16), F32),
        pltpu.VMEM((2560, 16), F32),
        pltpu.VMEM_SHARED((NBP, 16), F32),
    ],
    compiler_params=_SC_PARAMS,
)


NGRP = 80
NSUB = 640


def _msg_body(y0, y1, y2, y3, src_hbm, dst_hbm, zs_hbm,
              m0, m1, m2, m3, sidx, didx, rows, vout, acc,
              is0, is1, is2, is3,
              b0, b1, b2, b3, b4, b5, b6, b7):
    c = lax.axis_index("c")
    s = lax.axis_index("s")
    base = s * 2560
    ylist = (y0, y1, y2, y3)
    mlist = (m0, m1, m2, m3)
    isems = (is0, is1, is2, is3)
    bsems = (b0, b1, b2, b3, b4, b5, b6, b7)

    for p in range(2):
        for cv in range(2):

            @pl.when(c == cv)
            def _(p=p, cv=cv):
                ysel = ylist[2 * p + cv]
                msel = mlist[2 * p + cv]

                pltpu.sync_copy(zs_hbm, vout)
                for h in range(16):
                    pltpu.sync_copy(vout, acc.at[pl.ds(base + h * 160, 160)])
                plsc.subcore_barrier()

                tb = s * NGRP

                @pl.loop(0, NGRP)
                def _(g):
                    grp = tb + g
                    pltpu.sync_copy(src_hbm.at[grp], sidx.at[0])
                    pltpu.sync_copy(dst_hbm.at[grp], didx.at[0])
                    descs = [
                        pltpu.async_copy(ysel.at[sidx.at[0, b]], rows.at[b],
                                         bsems[0])
                        for b in range(8)
                    ]
                    for dsc in descs:
                        dsc.wait()
                    for b in range(8):
                        pltpu.sync_copy(rows.at[b], acc.at[didx.at[0, b]],
                                        add=True)

                plsc.subcore_barrier()
                for h in range(16):
                    pltpu.sync_copy(acc.at[pl.ds(base + h * 160, 160)], vout)
                    pltpu.sync_copy(vout, msel.at[pl.ds(base + h * 160, 160)])


_msg_call = pl.kernel(
    _msg_body,
    out_type=tuple(jax.ShapeDtypeStruct((NBP, 32), F32) for _ in range(4)),
    mesh=_SC_MESH,
    scratch_types=[
        pltpu.VMEM((4, 8, 128), jnp.int32),
        pltpu.VMEM((4, 8, 128), jnp.int32),
        pltpu.VMEM((8, 128, 32), F32),
        pltpu.VMEM((160, 32), F32),
        pltpu.VMEM_SHARED((NBP, 32), F32),
        pltpu.SemaphoreType.DMA,
        pltpu.SemaphoreType.DMA,
        pltpu.SemaphoreType.DMA,
        pltpu.SemaphoreType.DMA,
        pltpu.SemaphoreType.DMA,
        pltpu.SemaphoreType.DMA,
        pltpu.SemaphoreType.DMA,
        pltpu.SemaphoreType.DMA,
        pltpu.SemaphoreType.DMA,
        pltpu.SemaphoreType.DMA,
        pltpu.SemaphoreType.DMA,
        pltpu.SemaphoreType.DMA,
    ],
    compiler_params=_SC_PARAMS,
)


def _mid_body(x0_ref, wc_ref, deg_ref, bc_ref, y0, y1, y2, y3, z_ref):
    d = deg_ref[0, :, 0:1] + deg_ref[1, :, 0:1] + 1.0
    dinv = lax.rsqrt(d)
    xw = jnp.dot(x0_ref[...], wc_ref[...], preferred_element_type=F32)
    y = xw * dinv
    z_ref[...] = y * dinv + bc_ref[...]
    y0[...] = y[:, 0:32]
    y1[...] = y[:, 32:64]
    y2[...] = y[:, 64:96]
    y3[...] = y[:, 96:128]


_mid_call = pl.pallas_call(
    _mid_body,
    grid=(20,),
    in_specs=[
        pl.BlockSpec((2000, D), lambda i: (i, 0)),
        pl.BlockSpec((D, D), lambda i: (0, 0)),
        pl.BlockSpec((2, 2000, 16), lambda i: (0, i, 0)),
        pl.BlockSpec((1, D), lambda i: (0, 0)),
    ],
    out_specs=[
        pl.BlockSpec((2000, 32), lambda i: (i, 0)),
        pl.BlockSpec((2000, 32), lambda i: (i, 0)),
        pl.BlockSpec((2000, 32), lambda i: (i, 0)),
        pl.BlockSpec((2000, 32), lambda i: (i, 0)),
        pl.BlockSpec((2000, D), lambda i: (i, 0)),
    ],
    out_shape=[
        jax.ShapeDtypeStruct((NBP, 32), F32),
        jax.ShapeDtypeStruct((NBP, 32), F32),
        jax.ShapeDtypeStruct((NBP, 32), F32),
        jax.ShapeDtypeStruct((NBP, 32), F32),
        jax.ShapeDtypeStruct((NB, D), F32),
    ],
)


def _leaky(x):
    return jnp.where(x >= 0, x, 0.01 * x)


def _epi_body(m0, m1, m2, m3, z_ref, deg_ref, x0_ref,
              w1_ref, b1_ref, w2_ref, b2_ref, o_ref):
    d = deg_ref[0, :, 0:1] + deg_ref[1, :, 0:1] + 1.0
    dinv = lax.rsqrt(d)
    msg = jnp.concatenate([m0[...], m1[...], m2[...], m3[...]], axis=1)
    h = jnp.maximum(msg * dinv + z_ref[...], 0.0)
    x1 = h + x0_ref[...]
    a = jnp.dot(x1, w1_ref[...], preferred_element_type=F32) + b1_ref[...]
    a = _leaky(a)
    o = jnp.dot(a, w2_ref[...], preferred_element_type=F32) + b2_ref[...]
    o_ref[...] = _leaky(o)


_epi_call = pl.pallas_call(
    _epi_body,
    grid=(20,),
    in_specs=[
        pl.BlockSpec((2000, 32), lambda i: (i, 0)),
        pl.BlockSpec((2000, 32), lambda i: (i, 0)),
        pl.BlockSpec((2000, 32), lambda i: (i, 0)),
        pl.BlockSpec((2000, 32), lambda i: (i, 0)),
        pl.BlockSpec((2000, D), lambda i: (i, 0)),
        pl.BlockSpec((2, 2000, 16), lambda i: (0, i, 0)),
        pl.BlockSpec((2000, D), lambda i: (i, 0)),
        pl.BlockSpec((D, D), lambda i: (0, 0)),
        pl.BlockSpec((1, D), lambda i: (0, 0)),
        pl.BlockSpec((D, D), lambda i: (0, 0)),
        pl.BlockSpec((1, D), lambda i: (0, 0)),
    ],
    out_specs=pl.BlockSpec((2000, D), lambda i: (i, 0)),
    out_shape=jax.ShapeDtypeStruct((NB, D), F32),
)


def kernel(node_features, edge_index, Wc, bc, W1, b1, W2, b2):
    b_, n_, d_ = node_features.shape
    x0 = node_features.reshape(b_ * n_, d_)
    off = (jnp.arange(b_, dtype=edge_index.dtype) * n_)[:, None]
    npad = TEP - TE
    src = jnp.concatenate([
        (edge_index[:, 0, :] + off).reshape(-1),
        jnp.zeros((npad,), edge_index.dtype)])
    dst = jnp.concatenate([
        (edge_index[:, 1, :] + off).reshape(-1),
        jnp.full((npad,), PAD_DST, edge_index.dtype)])

    ones_c = jnp.ones((1024, 16), F32)
    zs_d = jnp.zeros((2560, 16), F32)
    zs_m = jnp.zeros((160, 32), F32)

    src3 = src.reshape(1280, 8, 128)
    dst3 = dst.reshape(1280, 8, 128)

    deg = _deg_call(dst, ones_c, zs_d)
    y0, y1, y2, y3, z = _mid_call(x0, Wc, deg, bc.reshape(1, d_))
    m0, m1, m2, m3 = _msg_call(y0, y1, y2, y3, src3, dst3, zs_m)
    out = _epi_call(m0, m1, m2, m3, z, deg, x0,
                    W1, b1.reshape(1, -1), W2, b2.reshape(1, -1))
    return out.reshape(b_, n_, -1)


# speedup vs baseline: 1.7416x; 1.7416x over previous
import jax
import jax.numpy as jnp
from jax import lax
from jax.experimental import pallas as pl
from jax.experimental.pallas import tpu as pltpu
from jax.experimental.pallas import tpu_sc as plsc

F32 = jnp.float32

NB = 40000
NBP = 40960
D = 128
NBLK = 10000

_SC_MESH = plsc.VectorSubcoreMesh(core_axis_name="c", subcore_axis_name="s")
_SC_PARAMS = pltpu.CompilerParams(use_tc_tiling_on_sc=False)


def _deg_body(dst_hbm, ones_hbm, zs_hbm, deg_hbm, didx, ones_v, vbuf, acc):
    c = lax.axis_index("c")
    s = lax.axis_index("s")
    base = s * 2560
    pltpu.sync_copy(ones_hbm, ones_v)
    pltpu.sync_copy(zs_hbm, vbuf)
    pltpu.sync_copy(vbuf, acc.at[pl.ds(base, 2560)])
    plsc.subcore_barrier()

    @pl.loop(0, 40)
    def _(i):
        g = i * 16 + s

        @pl.when(g < 625)
        def _():
            blk0 = (c * 625 + g) * 8
            pltpu.sync_copy(dst_hbm.at[pl.ds(blk0, 8)], didx)
            for b in range(8):
                pltpu.sync_copy(ones_v, acc.at[didx.at[b]], add=True)

    plsc.subcore_barrier()
    pltpu.sync_copy(acc.at[pl.ds(base, 2560)], vbuf)
    pltpu.sync_copy(vbuf, deg_hbm.at[c, pl.ds(base, 2560)])


_deg_call = pl.kernel(
    _deg_body,
    out_type=jax.ShapeDtypeStruct((2, NBP, 16), F32),
    mesh=_SC_MESH,
    scratch_types=[
        pltpu.VMEM((8, 128), jnp.int32),
        pltpu.VMEM((128, 16), F32),
        pltpu.VMEM((2560, 16), F32),
        pltpu.VMEM_SHARED((NBP, 16), F32),
    ],
    compiler_params=_SC_PARAMS,
)


def _msg_body(y0, y1, y2, y3, src_hbm, dst_hbm, zs_hbm,
              m0, m1, m2, m3, sidx, didx, rows, vout, acc, gsem):
    c = lax.axis_index("c")
    s = lax.axis_index("s")
    base = s * 2560
    ylist = (y0, y1, y2, y3)
    mlist = (m0, m1, m2, m3)

    for p in range(2):
        for cv in range(2):

            @pl.when(c == cv)
            def _(p=p, cv=cv):
                ysel = ylist[2 * p + cv]
                msel = mlist[2 * p + cv]
                pltpu.sync_copy(zs_hbm, vout)
                for h in range(10):
                    pltpu.sync_copy(vout, acc.at[pl.ds(base + h * 256, 256)])
                plsc.subcore_barrier()

                @pl.loop(0, 79)
                def _(i):
                    g = i * 16 + s

                    @pl.when(g < 1250)
                    def _():
                        blk0 = g * 8
                        pltpu.sync_copy(src_hbm.at[pl.ds(blk0, 8)], sidx)
                        pltpu.sync_copy(dst_hbm.at[pl.ds(blk0, 8)], didx)
                        descs = [
                            pltpu.async_copy(ysel.at[sidx.at[b]], rows.at[b],
                                             gsem)
                            for b in range(8)
                        ]
                        for dsc in descs:
                            dsc.wait()
                        for b in range(8):
                            pltpu.sync_copy(rows.at[b], acc.at[didx.at[b]],
                                            add=True)

                plsc.subcore_barrier()
                for h in range(10):
                    pltpu.sync_copy(acc.at[pl.ds(base + h * 256, 256)], vout)
                    pltpu.sync_copy(vout, msel.at[pl.ds(base + h * 256, 256)])


_msg_call = pl.kernel(
    _msg_body,
    out_type=tuple(jax.ShapeDtypeStruct((NBP, 32), F32) for _ in range(4)),
    mesh=_SC_MESH,
    scratch_types=[
        pltpu.VMEM((8, 128), jnp.int32),
        pltpu.VMEM((8, 128), jnp.int32),
        pltpu.VMEM((8, 128, 32), F32),
        pltpu.VMEM((256, 32), F32),
        pltpu.VMEM_SHARED((NBP, 32), F32),
        pltpu.SemaphoreType.DMA,
    ],
    compiler_params=_SC_PARAMS,
)


def _mm_body(x_ref, w_ref, o_ref):
    o_ref[...] = jnp.dot(x_ref[...], w_ref[...],
                         preferred_element_type=F32)


_mm_call = pl.pallas_call(
    _mm_body,
    grid=(20,),
    in_specs=[
        pl.BlockSpec((2000, D), lambda i: (i, 0)),
        pl.BlockSpec((D, D), lambda i: (0, 0)),
    ],
    out_specs=pl.BlockSpec((2000, D), lambda i: (i, 0)),
    out_shape=jax.ShapeDtypeStruct((NB, D), F32),
)


def _mid_body(xw_ref, deg_ref, bc_ref, y0, y1, y2, y3, z_ref):
    d = deg_ref[0, :, 0:1] + deg_ref[1, :, 0:1] + 1.0
    dinv = lax.rsqrt(d)
    xw = xw_ref[...]
    y = xw * dinv
    z_ref[...] = y * dinv + bc_ref[...]
    y0[...] = y[:, 0:32]
    y1[...] = y[:, 32:64]
    y2[...] = y[:, 64:96]
    y3[...] = y[:, 96:128]


_mid_call = pl.pallas_call(
    _mid_body,
    grid=(20,),
    in_specs=[
        pl.BlockSpec((2000, D), lambda i: (i, 0)),
        pl.BlockSpec((2, 2000, 16), lambda i: (0, i, 0)),
        pl.BlockSpec((1, D), lambda i: (0, 0)),
    ],
    out_specs=[
        pl.BlockSpec((2000, 32), lambda i: (i, 0)),
        pl.BlockSpec((2000, 32), lambda i: (i, 0)),
        pl.BlockSpec((2000, 32), lambda i: (i, 0)),
        pl.BlockSpec((2000, 32), lambda i: (i, 0)),
        pl.BlockSpec((2000, D), lambda i: (i, 0)),
    ],
    out_shape=[
        jax.ShapeDtypeStruct((NBP, 32), F32),
        jax.ShapeDtypeStruct((NBP, 32), F32),
        jax.ShapeDtypeStruct((NBP, 32), F32),
        jax.ShapeDtypeStruct((NBP, 32), F32),
        jax.ShapeDtypeStruct((NB, D), F32),
    ],
)


def _leaky(x):
    return jnp.where(x >= 0, x, 0.01 * x)


def _epi_body(m0, m1, m2, m3, z_ref, deg_ref, x0_ref,
              w1_ref, b1_ref, w2_ref, b2_ref, o_ref):
    d = deg_ref[0, :, 0:1] + deg_ref[1, :, 0:1] + 1.0
    dinv = lax.rsqrt(d)
    msg = jnp.concatenate([m0[...], m1[...], m2[...], m3[...]], axis=1)
    h = jnp.maximum(msg * dinv + z_ref[...], 0.0)
    x1 = h + x0_ref[...]
    a = jnp.dot(x1, w1_ref[...], preferred_element_type=F32) + b1_ref[...]
    a = _leaky(a)
    o = jnp.dot(a, w2_ref[...], preferred_element_type=F32) + b2_ref[...]
    o_ref[...] = _leaky(o)


_epi_call = pl.pallas_call(
    _epi_body,
    grid=(20,),
    in_specs=[
        pl.BlockSpec((2000, 32), lambda i: (i, 0)),
        pl.BlockSpec((2000, 32), lambda i: (i, 0)),
        pl.BlockSpec((2000, 32), lambda i: (i, 0)),
        pl.BlockSpec((2000, 32), lambda i: (i, 0)),
        pl.BlockSpec((2000, D), lambda i: (i, 0)),
        pl.BlockSpec((2, 2000, 16), lambda i: (0, i, 0)),
        pl.BlockSpec((2000, D), lambda i: (i, 0)),
        pl.BlockSpec((D, D), lambda i: (0, 0)),
        pl.BlockSpec((1, D), lambda i: (0, 0)),
        pl.BlockSpec((D, D), lambda i: (0, 0)),
        pl.BlockSpec((1, D), lambda i: (0, 0)),
    ],
    out_specs=pl.BlockSpec((2000, D), lambda i: (i, 0)),
    out_shape=jax.ShapeDtypeStruct((NB, D), F32),
)


def kernel(node_features, edge_index, Wc, bc, W1, b1, W2, b2):
    b_, n_, d_ = node_features.shape
    x0 = node_features.reshape(b_ * n_, d_)
    off = (jnp.arange(b_, dtype=edge_index.dtype) * n_)[:, None]
    src = (edge_index[:, 0, :] + off).reshape(NBLK, 128)
    dst = (edge_index[:, 1, :] + off).reshape(NBLK, 128)

    ones_c = jnp.ones((128, 16), F32)
    zs_d = jnp.zeros((2560, 16), F32)
    zs_m = jnp.zeros((256, 32), F32)

    deg = _deg_call(dst, ones_c, zs_d)
    xw = _mm_call(x0, Wc)
    y0, y1, y2, y3, z = _mid_call(xw, deg, bc.reshape(1, d_))
    m0, m1, m2, m3 = _msg_call(y0, y1, y2, y3, src, dst, zs_m)
    out = _epi_call(m0, m1, m2, m3, z, deg, x0,
                    W1, b1.reshape(1, -1), W2, b2.reshape(1, -1))
    return out.reshape(b_, n_, -1)


# speedup vs baseline: 1.7475x; 1.0034x over previous
import jax
import jax.numpy as jnp
from jax import lax
from jax.experimental import pallas as pl
from jax.experimental.pallas import tpu as pltpu
from jax.experimental.pallas import tpu_sc as plsc

F32 = jnp.float32

NB = 40000
NBP = 40960
D = 128
NBLK = 10000

_SC_MESH = plsc.VectorSubcoreMesh(core_axis_name="c", subcore_axis_name="s")
_SC_PARAMS = pltpu.CompilerParams(use_tc_tiling_on_sc=False)


def _deg_body(dst_hbm, ones_hbm, zs_hbm, deg_hbm, didx, ones_v, vbuf, acc):
    c = lax.axis_index("c")
    s = lax.axis_index("s")
    base = s * 2560
    pltpu.sync_copy(ones_hbm, ones_v)
    pltpu.sync_copy(zs_hbm, vbuf)
    pltpu.sync_copy(vbuf, acc.at[pl.ds(base, 2560)])
    plsc.subcore_barrier()

    @pl.loop(0, 40)
    def _(i):
        g = i * 16 + s

        @pl.when(g < 625)
        def _():
            blk0 = (c * 625 + g) * 8
            pltpu.sync_copy(dst_hbm.at[pl.ds(blk0, 8)], didx)
            for b in range(8):
                pltpu.sync_copy(ones_v, acc.at[didx.at[b]], add=True)

    plsc.subcore_barrier()
    pltpu.sync_copy(acc.at[pl.ds(base, 2560)], vbuf)
    pltpu.sync_copy(vbuf, deg_hbm.at[c, pl.ds(base, 2560)])


_deg_call = pl.kernel(
    _deg_body,
    out_type=jax.ShapeDtypeStruct((2, NBP, 16), F32),
    mesh=_SC_MESH,
    scratch_types=[
        pltpu.VMEM((8, 128), jnp.int32),
        pltpu.VMEM((128, 16), F32),
        pltpu.VMEM((2560, 16), F32),
        pltpu.VMEM_SHARED((NBP, 16), F32),
    ],
    compiler_params=_SC_PARAMS,
)


def _msg_body(y0, y1, y2, y3, src_hbm, dst_hbm, zs_hbm,
              m0, m1, m2, m3, sidx, didx, rows, vout, acc, gsem):
    c = lax.axis_index("c")
    s = lax.axis_index("s")
    base = s * 2560
    ylist = (y0, y1, y2, y3)
    mlist = (m0, m1, m2, m3)

    for p in range(2):
        for cv in range(2):

            @pl.when(c == cv)
            def _(p=p, cv=cv):
                ysel = ylist[2 * p + cv]
                msel = mlist[2 * p + cv]
                pltpu.sync_copy(zs_hbm, acc.at[pl.ds(base, 2560)])
                plsc.subcore_barrier()

                @pl.loop(0, 79)
                def _(i):
                    g = i * 16 + s

                    @pl.when(g < 1250)
                    def _():
                        blk0 = g * 8
                        pltpu.sync_copy(src_hbm.at[pl.ds(blk0, 8)], sidx)
                        pltpu.sync_copy(dst_hbm.at[pl.ds(blk0, 8)], didx)
                        descs = [
                            pltpu.async_copy(ysel.at[sidx.at[b]], rows.at[b],
                                             gsem)
                            for b in range(8)
                        ]
                        for dsc in descs:
                            dsc.wait()
                        for b in range(8):
                            pltpu.sync_copy(rows.at[b], acc.at[didx.at[b]],
                                            add=True)

                plsc.subcore_barrier()
                pltpu.sync_copy(acc.at[pl.ds(base, 2560)],
                                msel.at[pl.ds(base, 2560)])


_msg_call = pl.kernel(
    _msg_body,
    out_type=tuple(jax.ShapeDtypeStruct((NBP, 32), F32) for _ in range(4)),
    mesh=_SC_MESH,
    scratch_types=[
        pltpu.VMEM((8, 128), jnp.int32),
        pltpu.VMEM((8, 128), jnp.int32),
        pltpu.VMEM((8, 128, 32), F32),
        pltpu.VMEM((256, 32), F32),
        pltpu.VMEM_SHARED((NBP, 32), F32),
        pltpu.SemaphoreType.DMA,
    ],
    compiler_params=_SC_PARAMS,
)


def _mm_body(x_ref, w_ref, o_ref):
    o_ref[...] = jnp.dot(x_ref[...], w_ref[...],
                         preferred_element_type=F32)


_mm_call = pl.pallas_call(
    _mm_body,
    grid=(20,),
    in_specs=[
        pl.BlockSpec((2000, D), lambda i: (i, 0)),
        pl.BlockSpec((D, D), lambda i: (0, 0)),
    ],
    out_specs=pl.BlockSpec((2000, D), lambda i: (i, 0)),
    out_shape=jax.ShapeDtypeStruct((NB, D), F32),
)


def _mid_body(xw_ref, deg_ref, bc_ref, y0, y1, y2, y3, z_ref):
    d = deg_ref[0, :, 0:1] + deg_ref[1, :, 0:1] + 1.0
    dinv = lax.rsqrt(d)
    xw = xw_ref[...]
    y = xw * dinv
    z_ref[...] = y * dinv + bc_ref[...]
    y0[...] = y[:, 0:32]
    y1[...] = y[:, 32:64]
    y2[...] = y[:, 64:96]
    y3[...] = y[:, 96:128]


_mid_call = pl.pallas_call(
    _mid_body,
    grid=(20,),
    in_specs=[
        pl.BlockSpec((2000, D), lambda i: (i, 0)),
        pl.BlockSpec((2, 2000, 16), lambda i: (0, i, 0)),
        pl.BlockSpec((1, D), lambda i: (0, 0)),
    ],
    out_specs=[
        pl.BlockSpec((2000, 32), lambda i: (i, 0)),
        pl.BlockSpec((2000, 32), lambda i: (i, 0)),
        pl.BlockSpec((2000, 32), lambda i: (i, 0)),
        pl.BlockSpec((2000, 32), lambda i: (i, 0)),
        pl.BlockSpec((2000, D), lambda i: (i, 0)),
    ],
    out_shape=[
        jax.ShapeDtypeStruct((NBP, 32), F32),
        jax.ShapeDtypeStruct((NBP, 32), F32),
        jax.ShapeDtypeStruct((NBP, 32), F32),
        jax.ShapeDtypeStruct((NBP, 32), F32),
        jax.ShapeDtypeStruct((NB, D), F32),
    ],
)


def _leaky(x):
    return jnp.where(x >= 0, x, 0.01 * x)


def _epi_body(m0, m1, m2, m3, z_ref, deg_ref, x0_ref,
              w1_ref, b1_ref, w2_ref, b2_ref, o_ref):
    d = deg_ref[0, :, 0:1] + deg_ref[1, :, 0:1] + 1.0
    dinv = lax.rsqrt(d)
    msg = jnp.concatenate([m0[...], m1[...], m2[...], m3[...]], axis=1)
    h = jnp.maximum(msg * dinv + z_ref[...], 0.0)
    x1 = h + x0_ref[...]
    a = jnp.dot(x1, w1_ref[...], preferred_element_type=F32) + b1_ref[...]
    a = _leaky(a)
    o = jnp.dot(a, w2_ref[...], preferred_element_type=F32) + b2_ref[...]
    o_ref[...] = _leaky(o)


_epi_call = pl.pallas_call(
    _epi_body,
    grid=(20,),
    in_specs=[
        pl.BlockSpec((2000, 32), lambda i: (i, 0)),
        pl.BlockSpec((2000, 32), lambda i: (i, 0)),
        pl.BlockSpec((2000, 32), lambda i: (i, 0)),
        pl.BlockSpec((2000, 32), lambda i: (i, 0)),
        pl.BlockSpec((2000, D), lambda i: (i, 0)),
        pl.BlockSpec((2, 2000, 16), lambda i: (0, i, 0)),
        pl.BlockSpec((2000, D), lambda i: (i, 0)),
        pl.BlockSpec((D, D), lambda i: (0, 0)),
        pl.BlockSpec((1, D), lambda i: (0, 0)),
        pl.BlockSpec((D, D), lambda i: (0, 0)),
        pl.BlockSpec((1, D), lambda i: (0, 0)),
    ],
    out_specs=pl.BlockSpec((2000, D), lambda i: (i, 0)),
    out_shape=jax.ShapeDtypeStruct((NB, D), F32),
)


def kernel(node_features, edge_index, Wc, bc, W1, b1, W2, b2):
    b_, n_, d_ = node_features.shape
    x0 = node_features.reshape(b_ * n_, d_)
    off = (jnp.arange(b_, dtype=edge_index.dtype) * n_)[:, None]
    src = (edge_index[:, 0, :] + off).reshape(NBLK, 128)
    dst = (edge_index[:, 1, :] + off).reshape(NBLK, 128)

    ones_c = jnp.ones((128, 16), F32)
    zs_d = jnp.zeros((2560, 16), F32)
    zs_m = jnp.zeros((2560, 32), F32)

    deg = _deg_call(dst, ones_c, zs_d)
    xw = _mm_call(x0, Wc)
    y0, y1, y2, y3, z = _mid_call(xw, deg, bc.reshape(1, d_))
    m0, m1, m2, m3 = _msg_call(y0, y1, y2, y3, src, dst, zs_m)
    out = _epi_call(m0, m1, m2, m3, z, deg, x0,
                    W1, b1.reshape(1, -1), W2, b2.reshape(1, -1))
    return out.reshape(b_, n_, -1)
